# baseline (speedup 1.0000x reference)
import math

import numpy as np
import jax
import jax.numpy as jnp
from jax.experimental import pallas as pl
from jax.experimental.pallas import tpu as pltpu

_K = 3
_J = 3
_NU = 0.001
_ALPHA = 0.1
_N = 784
_N1 = 196
_C0 = 512
_C1 = 1024
_CE = 256
_NC = 2048
_NCHUNK = 16
_W = _NC // _NCHUNK
_RB = 16


def _build_mup() -> np.ndarray:
    s = np.zeros((14, 14), np.float32)
    for i in range(14):
        for j in range(max(0, i - 1), min(14, i + 2)):
            s[i, j] = 1.0
    u = np.zeros((28, 14), np.float32)
    for i in range(28):
        c = (i + 0.5) / 2.0 - 0.5
        f = math.floor(c)
        w_hi = c - f
        for idx, w in ((f, 1.0 - w_hi), (f + 1, w_hi)):
            u[i, min(13, max(0, idx))] += w
    a = u @ s
    return (np.kron(a, a) / 9.0).astype(np.float32)


_MUP = _build_mup()


def _batcher_sort_ces(n):
    ces = []

    def merge(lo, n_, r):
        m = r * 2
        if m < n_:
            merge(lo, n_, m)
            merge(lo + r, n_, m)
            for i in range(lo + r, lo + n_ - r, m):
                ces.append((i, i + r))
        else:
            ces.append((lo, lo + r))

    def sort(lo, n_):
        if n_ > 1:
            h = n_ // 2
            sort(lo, h)
            sort(lo + h, h)
            merge(lo, n_, 1)

    sort(0, n)
    return ces


def _batcher_merge16_ces():
    ces = []

    def merge(lo, n_, r):
        m = r * 2
        if m < n_:
            merge(lo, n_, m)
            merge(lo + r, n_, m)
            for i in range(lo + r, lo + n_ - r, m):
                ces.append((i, i + r))
        else:
            ces.append((lo, lo + r))

    merge(0, 16, 1)
    return ces


def _ssa_prune(ces, init_sym, n_in, keep_outputs):
    inf = -1
    sym = list(init_sym)
    prog = []
    nxt = n_in
    for i, j in ces:
        a, b = sym[i], sym[j]
        if b == inf:
            continue
        if a == inf:
            sym[i], sym[j] = b, inf
            continue
        lo, hi = nxt, nxt + 1
        nxt += 2
        prog.append((a, b, lo, hi))
        sym[i], sym[j] = lo, hi
    outs = [sym[k] for k in keep_outputs]
    need = set(outs)
    kept = []
    for a, b, lo, hi in reversed(prog):
        if lo in need or hi in need:
            kept.append((a, b, lo, hi))
            need.update((a, b))
    return list(reversed(kept)), outs


_S16_PROG, _S16_OUT = _ssa_prune(
    _batcher_sort_ces(16), list(range(16)), 16, range(6))
_M66_PROG, _M66_OUT = _ssa_prune(
    _batcher_merge16_ces(),
    [0, 1, 2, 3, 4, 5, -1, -1, 6, 7, 8, 9, 10, 11, -1, -1], 12, range(6))


def _run_prog(prog, outs, env):
    for a, b, lo, hi in prog:
        x, y = env[a], env[b]
        env[lo] = jnp.minimum(x, y)
        env[hi] = jnp.maximum(x, y)
    return [env[o] for o in outs]


def _noad_kernel(x0_ref, x1_ref, w0_ref, w1_ref, mup_ref, ct_ref, b_ref,
                 r_ref, emb_ref, score_ref, loss_ref, cn_ref, v_ref, en_ref):
    b = pl.program_id(0)

    @pl.when(b == 0)
    def _():
        ct0 = ct_ref[...]
        cn_ref[...] = jnp.sum(ct0 * ct0, axis=0, keepdims=True)

    dn_t = (((0,), (0,)), ((), ()))
    e0 = jax.lax.dot_general(x0_ref[0], w0_ref[...], dn_t,
                             preferred_element_type=jnp.float32)
    e1 = jax.lax.dot_general(x1_ref[0], w1_ref[...], dn_t,
                             preferred_element_type=jnp.float32)

    widx = jax.lax.broadcasted_iota(jnp.int32, (_N, 1), 0) % 28
    ml = (widx != 0).astype(jnp.float32)
    mr = (widx != 27).astype(jnp.float32)
    z1 = jnp.zeros((1, _CE), jnp.float32)
    sw = e0 + jnp.concatenate([z1, e0[:-1, :]], axis=0) * ml \
            + jnp.concatenate([e0[1:, :], z1], axis=0) * mr
    z28 = jnp.zeros((28, _CE), jnp.float32)
    pooled0 = (sw + jnp.concatenate([z28, sw[:-28, :]], axis=0)
                  + jnp.concatenate([sw[28:, :], z28], axis=0)) * (1.0 / 9.0)

    e1u = jnp.dot(mup_ref[...], e1,
                  preferred_element_type=jnp.float32)
    et = pooled0 + e1u + b_ref[...]
    emb_ref[0] = et

    g = jnp.dot(et, ct_ref[...],
                preferred_element_type=jnp.float32)
    v_ref[...] = cn_ref[...] - 2.0 * g
    en_ref[...] = jnp.sum(et * et, axis=1, keepdims=True)

    r2 = r_ref[0, 0] * r_ref[0, 0]
    zero = jnp.float32(0.0)
    scale = 1.0 / (_NU * pl.num_programs(0) * _N * _K)

    def block_topk(i, acc):
        row0 = i * _RB
        vb = v_ref[pl.ds(row0, _RB), :]
        env = {k: vb[:, _W * k:_W * (k + 1)] for k in range(_NCHUNK)}
        lists = _run_prog(_S16_PROG, _S16_OUT, env)
        for s in (64, 32, 16, 8, 4, 2, 1):
            env = {i2: lists[i2] for i2 in range(6)}
            for i2 in range(6):
                env[6 + i2] = pltpu.roll(lists[i2], _W - s, 1)
            lists = _run_prog(_M66_PROG, _M66_OUT, env)
        enb = en_ref[pl.ds(row0, _RB), :]
        t0, t1, t2, t3, t4, t5 = [L[:, 0:1] + enb for L in lists]

        d0 = jnp.sqrt(jnp.maximum(t0, 1e-12))
        d1 = jnp.sqrt(jnp.maximum(t1, 1e-12))
        d2s = jnp.sqrt(jnp.maximum(t2, 1e-12))
        sm0 = 1.0 / (1.0 + jnp.exp(d0 - d1) + jnp.exp(d0 - d2s))
        score_ref[0, pl.ds(row0, _RB), :] = sm0 * d0

        s1 = (jnp.maximum(t0 - r2, zero) + jnp.maximum(t1 - r2, zero)
              + jnp.maximum(t2 - r2, zero))
        s2 = (jnp.maximum(r2 + _ALPHA - t3, zero)
              + jnp.maximum(r2 + _ALPHA - t4, zero)
              + jnp.maximum(r2 + _ALPHA - t5, zero))
        return acc + (jnp.sum(s1) + jnp.sum(s2)) * scale

    part = jax.lax.fori_loop(0, _N // _RB, block_topk, jnp.float32(0.0))

    @pl.when(b == 0)
    def _():
        loss_ref[...] = jnp.zeros_like(loss_ref)

    loss_ref[...] += part


def _run(x0, x1, w0, w1, mup, ct, b2, r2, interpret=False):
    batch = x0.shape[0]
    f32 = jnp.float32
    return pl.pallas_call(
        _noad_kernel,
        grid=(batch,),
        in_specs=[
            pl.BlockSpec((1, _C0, _N), lambda b: (b, 0, 0)),
            pl.BlockSpec((1, _C1, _N1), lambda b: (b, 0, 0)),
            pl.BlockSpec((_C0, _CE), lambda b: (0, 0)),
            pl.BlockSpec((_C1, _CE), lambda b: (0, 0)),
            pl.BlockSpec((_N, _N1), lambda b: (0, 0)),
            pl.BlockSpec((_CE, _NC), lambda b: (0, 0)),
            pl.BlockSpec((1, _CE), lambda b: (0, 0)),
            pl.BlockSpec((1, 1), lambda b: (0, 0)),
        ],
        out_specs=[
            pl.BlockSpec((1, _N, _CE), lambda b: (b, 0, 0)),
            pl.BlockSpec((1, _N, 1), lambda b: (b, 0, 0)),
            pl.BlockSpec((1, 1), lambda b: (0, 0)),
        ],
        out_shape=[
            jax.ShapeDtypeStruct((batch, _N, _CE), f32),
            jax.ShapeDtypeStruct((batch, _N, 1), f32),
            jax.ShapeDtypeStruct((1, 1), f32),
        ],
        scratch_shapes=[pltpu.VMEM((1, _NC), f32),
                        pltpu.VMEM((_N, _NC), f32),
                        pltpu.VMEM((_N, 1), f32)],
        interpret=interpret,
    )(x0, x1, w0, w1, mup, ct, b2, r2)


def kernel(p0, p1, W_proj, b_proj, centroids, r):
    batch = p0.shape[0]
    x0 = p0.reshape(batch, _C0, _N)
    x1 = p1.reshape(batch, _C1, _N1)
    wt = W_proj.T
    embeds, score3, loss = _run(
        x0, x1, wt[:_C0], wt[_C0:], jnp.asarray(_MUP), centroids.T,
        b_proj.reshape(1, _CE), r.reshape(1, 1))
    score = score3.reshape(batch, 28, 28)[:, None, :, :]
    return (loss[0, 0], score, embeds)


# speedup vs baseline: 4.4922x; 4.4922x over previous
import math

import numpy as np
import jax
import jax.numpy as jnp
from jax.experimental import pallas as pl
from jax.experimental.pallas import tpu as pltpu

_K = 3
_J = 3
_NU = 0.001
_ALPHA = 0.1
_N = 784
_N1 = 196
_C0 = 512
_C1 = 1024
_CE = 256
_NC = 2048
_NCHUNK = 16
_W = _NC // _NCHUNK


def _build_mup() -> np.ndarray:
    s = np.zeros((14, 14), np.float32)
    for i in range(14):
        for j in range(max(0, i - 1), min(14, i + 2)):
            s[i, j] = 1.0
    u = np.zeros((28, 14), np.float32)
    for i in range(28):
        c = (i + 0.5) / 2.0 - 0.5
        f = math.floor(c)
        w_hi = c - f
        for idx, w in ((f, 1.0 - w_hi), (f + 1, w_hi)):
            u[i, min(13, max(0, idx))] += w
    a = u @ s
    return (np.kron(a, a) / 9.0).astype(np.float32)


_MUP = _build_mup()


def _batcher_sort_ces(n):
    ces = []

    def merge(lo, n_, r):
        m = r * 2
        if m < n_:
            merge(lo, n_, m)
            merge(lo + r, n_, m)
            for i in range(lo + r, lo + n_ - r, m):
                ces.append((i, i + r))
        else:
            ces.append((lo, lo + r))

    def sort(lo, n_):
        if n_ > 1:
            h = n_ // 2
            sort(lo, h)
            sort(lo + h, h)
            merge(lo, n_, 1)

    sort(0, n)
    return ces


def _batcher_merge16_ces():
    ces = []

    def merge(lo, n_, r):
        m = r * 2
        if m < n_:
            merge(lo, n_, m)
            merge(lo + r, n_, m)
            for i in range(lo + r, lo + n_ - r, m):
                ces.append((i, i + r))
        else:
            ces.append((lo, lo + r))

    merge(0, 16, 1)
    return ces


def _ssa_prune(ces, init_sym, n_in, keep_outputs):
    inf = -1
    sym = list(init_sym)
    prog = []
    nxt = n_in
    for i, j in ces:
        a, b = sym[i], sym[j]
        if b == inf:
            continue
        if a == inf:
            sym[i], sym[j] = b, inf
            continue
        lo, hi = nxt, nxt + 1
        nxt += 2
        prog.append((a, b, lo, hi))
        sym[i], sym[j] = lo, hi
    outs = [sym[k] for k in keep_outputs]
    need = set(outs)
    kept = []
    for a, b, lo, hi in reversed(prog):
        if lo in need or hi in need:
            kept.append((a, b, lo, hi))
            need.update((a, b))
    return list(reversed(kept)), outs


_S16_PROG, _S16_OUT = _ssa_prune(
    _batcher_sort_ces(16), list(range(16)), 16, range(6))
_M66_PROG, _M66_OUT = _ssa_prune(
    _batcher_merge16_ces(),
    [0, 1, 2, 3, 4, 5, -1, -1, 6, 7, 8, 9, 10, 11, -1, -1], 12, range(6))


def _run_prog(prog, outs, env):
    for a, b, lo, hi in prog:
        x, y = env[a], env[b]
        env[lo] = jnp.minimum(x, y)
        env[hi] = jnp.maximum(x, y)
    return [env[o] for o in outs]


def _noad_kernel(x0_ref, x1_ref, w0_ref, w1_ref, mup_ref, ct_ref, b_ref,
                 r_ref, emb_ref, score_ref, loss_ref, cn_ref):
    b = pl.program_id(0)

    @pl.when(b == 0)
    def _():
        ct0 = ct_ref[...]
        cn_ref[...] = jnp.sum(ct0 * ct0, axis=0, keepdims=True)

    dn_t = (((0,), (0,)), ((), ()))
    e0 = jax.lax.dot_general(x0_ref[0], w0_ref[...], dn_t,
                             preferred_element_type=jnp.float32)
    e1 = jax.lax.dot_general(x1_ref[0], w1_ref[...], dn_t,
                             preferred_element_type=jnp.float32)

    widx = jax.lax.broadcasted_iota(jnp.int32, (_N, 1), 0) % 28
    ml = (widx != 0).astype(jnp.float32)
    mr = (widx != 27).astype(jnp.float32)
    z1 = jnp.zeros((1, _CE), jnp.float32)
    sw = e0 + jnp.concatenate([z1, e0[:-1, :]], axis=0) * ml \
            + jnp.concatenate([e0[1:, :], z1], axis=0) * mr
    z28 = jnp.zeros((28, _CE), jnp.float32)
    pooled0 = (sw + jnp.concatenate([z28, sw[:-28, :]], axis=0)
                  + jnp.concatenate([sw[28:, :], z28], axis=0)) * (1.0 / 9.0)

    e1u = jnp.dot(mup_ref[...], e1,
                  preferred_element_type=jnp.float32)
    et = pooled0 + e1u + b_ref[...]
    emb_ref[0] = et

    g = jnp.dot(et, ct_ref[...],
                preferred_element_type=jnp.float32)
    v = cn_ref[...] - 2.0 * g
    en = jnp.sum(et * et, axis=1, keepdims=True)

    env = {k: v[:, _W * k:_W * (k + 1)] for k in range(_NCHUNK)}
    lists = _run_prog(_S16_PROG, _S16_OUT, env)
    for s in (64, 32):
        env = {i: lists[i] for i in range(6)}
        for i in range(6):
            env[6 + i] = pltpu.roll(lists[i], _W - s, 1)
        lists = _run_prog(_M66_PROG, _M66_OUT, env)
    lists = [jnp.transpose(L[:, 0:32]) for L in lists]
    for s in (16, 8, 4, 2, 1):
        env = {i: lists[i] for i in range(6)}
        for i in range(6):
            env[6 + i] = pltpu.roll(lists[i], 32 - s, 0)
        lists = _run_prog(_M66_PROG, _M66_OUT, env)
    en_row = jnp.transpose(en)
    t0, t1, t2, t3, t4, t5 = [L[0:1, :] + en_row for L in lists]

    d0 = jnp.sqrt(jnp.maximum(t0, 1e-12))
    d1 = jnp.sqrt(jnp.maximum(t1, 1e-12))
    d2s = jnp.sqrt(jnp.maximum(t2, 1e-12))
    sm0 = 1.0 / (1.0 + jnp.exp(d0 - d1) + jnp.exp(d0 - d2s))
    score_ref[0] = sm0 * d0

    r2 = r_ref[0, 0] * r_ref[0, 0]
    zero = jnp.float32(0.0)
    s1 = (jnp.maximum(t0 - r2, zero) + jnp.maximum(t1 - r2, zero)
          + jnp.maximum(t2 - r2, zero))
    s2 = (jnp.maximum(r2 + _ALPHA - t3, zero)
          + jnp.maximum(r2 + _ALPHA - t4, zero)
          + jnp.maximum(r2 + _ALPHA - t5, zero))
    scale = 1.0 / (_NU * pl.num_programs(0) * _N * _K)
    part = (jnp.sum(s1) + jnp.sum(s2)) * scale

    @pl.when(b == 0)
    def _():
        loss_ref[...] = jnp.zeros_like(loss_ref)

    loss_ref[...] += part


def _run(x0, x1, w0, w1, mup, ct, b2, r2, interpret=False):
    batch = x0.shape[0]
    f32 = jnp.float32
    return pl.pallas_call(
        _noad_kernel,
        grid=(batch,),
        in_specs=[
            pl.BlockSpec((1, _C0, _N), lambda b: (b, 0, 0)),
            pl.BlockSpec((1, _C1, _N1), lambda b: (b, 0, 0)),
            pl.BlockSpec((_C0, _CE), lambda b: (0, 0)),
            pl.BlockSpec((_C1, _CE), lambda b: (0, 0)),
            pl.BlockSpec((_N, _N1), lambda b: (0, 0)),
            pl.BlockSpec((_CE, _NC), lambda b: (0, 0)),
            pl.BlockSpec((1, _CE), lambda b: (0, 0)),
            pl.BlockSpec((1, 1), lambda b: (0, 0)),
        ],
        out_specs=[
            pl.BlockSpec((1, _N, _CE), lambda b: (b, 0, 0)),
            pl.BlockSpec((1, 1, _N), lambda b: (b, 0, 0)),
            pl.BlockSpec((1, 1), lambda b: (0, 0)),
        ],
        out_shape=[
            jax.ShapeDtypeStruct((batch, _N, _CE), f32),
            jax.ShapeDtypeStruct((batch, 1, _N), f32),
            jax.ShapeDtypeStruct((1, 1), f32),
        ],
        scratch_shapes=[pltpu.VMEM((1, _NC), f32)],
        interpret=interpret,
    )(x0, x1, w0, w1, mup, ct, b2, r2)


def kernel(p0, p1, W_proj, b_proj, centroids, r):
    batch = p0.shape[0]
    x0 = p0.reshape(batch, _C0, _N)
    x1 = p1.reshape(batch, _C1, _N1)
    wt = W_proj.T
    embeds, score3, loss = _run(
        x0, x1, wt[:_C0], wt[_C0:], jnp.asarray(_MUP), centroids.T,
        b_proj.reshape(1, _CE), r.reshape(1, 1))
    score = score3.reshape(batch, 28, 28)[:, None, :, :]
    return (loss[0, 0], score, embeds)


# speedup vs baseline: 4.5772x; 1.0189x over previous
import math

import numpy as np
import jax
import jax.numpy as jnp
from jax.experimental import pallas as pl
from jax.experimental.pallas import tpu as pltpu

_K = 3
_J = 3
_NU = 0.001
_ALPHA = 0.1
_N = 784
_N1 = 196
_C0 = 512
_C1 = 1024
_CE = 256
_NC = 2048
_NCHUNK = 16
_W = _NC // _NCHUNK


def _build_mup() -> np.ndarray:
    s = np.zeros((14, 14), np.float32)
    for i in range(14):
        for j in range(max(0, i - 1), min(14, i + 2)):
            s[i, j] = 1.0
    u = np.zeros((28, 14), np.float32)
    for i in range(28):
        c = (i + 0.5) / 2.0 - 0.5
        f = math.floor(c)
        w_hi = c - f
        for idx, w in ((f, 1.0 - w_hi), (f + 1, w_hi)):
            u[i, min(13, max(0, idx))] += w
    a = u @ s
    return (np.kron(a, a) / 9.0).astype(np.float32)


_MUP = _build_mup()


def _batcher_sort_ces(n):
    ces = []

    def merge(lo, n_, r):
        m = r * 2
        if m < n_:
            merge(lo, n_, m)
            merge(lo + r, n_, m)
            for i in range(lo + r, lo + n_ - r, m):
                ces.append((i, i + r))
        else:
            ces.append((lo, lo + r))

    def sort(lo, n_):
        if n_ > 1:
            h = n_ // 2
            sort(lo, h)
            sort(lo + h, h)
            merge(lo, n_, 1)

    sort(0, n)
    return ces


def _batcher_merge16_ces():
    ces = []

    def merge(lo, n_, r):
        m = r * 2
        if m < n_:
            merge(lo, n_, m)
            merge(lo + r, n_, m)
            for i in range(lo + r, lo + n_ - r, m):
                ces.append((i, i + r))
        else:
            ces.append((lo, lo + r))

    merge(0, 16, 1)
    return ces


def _ssa_prune(ces, init_sym, n_in, keep_outputs):
    inf = -1
    sym = list(init_sym)
    prog = []
    nxt = n_in
    for i, j in ces:
        a, b = sym[i], sym[j]
        if b == inf:
            continue
        if a == inf:
            sym[i], sym[j] = b, inf
            continue
        lo, hi = nxt, nxt + 1
        nxt += 2
        prog.append((a, b, lo, hi))
        sym[i], sym[j] = lo, hi
    outs = [sym[k] for k in keep_outputs]
    need = set(outs)
    kept = []
    for a, b, lo, hi in reversed(prog):
        if lo in need or hi in need:
            kept.append((a, b, lo, hi))
            need.update((a, b))
    return list(reversed(kept)), outs


_S16_PROG, _S16_OUT = _ssa_prune(
    _batcher_sort_ces(16), list(range(16)), 16, range(6))
_M66_PROG, _M66_OUT = _ssa_prune(
    _batcher_merge16_ces(),
    [0, 1, 2, 3, 4, 5, -1, -1, 6, 7, 8, 9, 10, 11, -1, -1], 12, range(6))


def _run_prog(prog, outs, env):
    for a, b, lo, hi in prog:
        x, y = env[a], env[b]
        env[lo] = jnp.minimum(x, y)
        env[hi] = jnp.maximum(x, y)
    return [env[o] for o in outs]


def _noad_kernel(x0_ref, x1_ref, w0_ref, w1_ref, mup_ref, ct_ref, b_ref,
                 r_ref, emb_ref, score_ref, loss_ref, cn_ref):
    b = pl.program_id(0)

    @pl.when(b == 0)
    def _():
        ct0 = ct_ref[...]
        cn_ref[...] = jax.lax.dot_general(
            jnp.ones((1, _CE), jnp.float32), ct0 * ct0,
            (((1,), (1,)), ((), ())), preferred_element_type=jnp.float32)

    dn_t = (((0,), (1,)), ((), ()))
    e0 = jax.lax.dot_general(x0_ref[0], w0_ref[...], dn_t,
                             preferred_element_type=jnp.float32)
    e1 = jax.lax.dot_general(x1_ref[0], w1_ref[...], dn_t,
                             preferred_element_type=jnp.float32)

    widx = jax.lax.broadcasted_iota(jnp.int32, (_N, 1), 0) % 28
    ml = (widx != 0).astype(jnp.float32)
    mr = (widx != 27).astype(jnp.float32)
    z1 = jnp.zeros((1, _CE), jnp.float32)
    sw = e0 + jnp.concatenate([z1, e0[:-1, :]], axis=0) * ml \
            + jnp.concatenate([e0[1:, :], z1], axis=0) * mr
    z28 = jnp.zeros((28, _CE), jnp.float32)
    pooled0 = (sw + jnp.concatenate([z28, sw[:-28, :]], axis=0)
                  + jnp.concatenate([sw[28:, :], z28], axis=0)) * (1.0 / 9.0)

    e1u = jnp.dot(mup_ref[...], e1,
                  preferred_element_type=jnp.float32)
    et = pooled0 + e1u + b_ref[...]
    emb_ref[0] = et

    g = jax.lax.dot_general(et, ct_ref[...], (((1,), (1,)), ((), ())),
                            preferred_element_type=jnp.float32)
    v = cn_ref[...] - 2.0 * g
    en = jnp.sum(et * et, axis=1, keepdims=True)

    env = {k: v[:, _W * k:_W * (k + 1)] for k in range(_NCHUNK)}
    lists = _run_prog(_S16_PROG, _S16_OUT, env)
    for s in (64, 32):
        env = {i: lists[i] for i in range(6)}
        for i in range(6):
            env[6 + i] = pltpu.roll(lists[i], _W - s, 1)
        lists = _run_prog(_M66_PROG, _M66_OUT, env)
    lists = [jnp.transpose(L[:, 0:32]) for L in lists]
    for s in (16, 8, 4, 2, 1):
        env = {i: lists[i] for i in range(6)}
        for i in range(6):
            env[6 + i] = pltpu.roll(lists[i], 32 - s, 0)
        lists = _run_prog(_M66_PROG, _M66_OUT, env)
    en_row = jnp.transpose(en)
    t0, t1, t2, t3, t4, t5 = [L[0:1, :] + en_row for L in lists]

    d0 = jnp.sqrt(jnp.maximum(t0, 1e-12))
    d1 = jnp.sqrt(jnp.maximum(t1, 1e-12))
    d2s = jnp.sqrt(jnp.maximum(t2, 1e-12))
    sm0 = 1.0 / (1.0 + jnp.exp(d0 - d1) + jnp.exp(d0 - d2s))
    score_ref[0] = sm0 * d0

    r2 = r_ref[0, 0] * r_ref[0, 0]
    zero = jnp.float32(0.0)
    s1 = (jnp.maximum(t0 - r2, zero) + jnp.maximum(t1 - r2, zero)
          + jnp.maximum(t2 - r2, zero))
    s2 = (jnp.maximum(r2 + _ALPHA - t3, zero)
          + jnp.maximum(r2 + _ALPHA - t4, zero)
          + jnp.maximum(r2 + _ALPHA - t5, zero))
    scale = 1.0 / (_NU * pl.num_programs(0) * _N * _K)
    part = (jnp.sum(s1) + jnp.sum(s2)) * scale

    @pl.when(b == 0)
    def _():
        loss_ref[...] = jnp.zeros_like(loss_ref)

    loss_ref[...] += part


def _run(x0, x1, w0, w1, mup, ct, b2, r2, interpret=False):
    batch = x0.shape[0]
    f32 = jnp.float32
    return pl.pallas_call(
        _noad_kernel,
        grid=(batch,),
        in_specs=[
            pl.BlockSpec((1, _C0, _N), lambda b: (b, 0, 0)),
            pl.BlockSpec((1, _C1, _N1), lambda b: (b, 0, 0)),
            pl.BlockSpec((_CE, _C0), lambda b: (0, 0)),
            pl.BlockSpec((_CE, _C1), lambda b: (0, 0)),
            pl.BlockSpec((_N, _N1), lambda b: (0, 0)),
            pl.BlockSpec((_NC, _CE), lambda b: (0, 0)),
            pl.BlockSpec((1, _CE), lambda b: (0, 0)),
            pl.BlockSpec((1, 1), lambda b: (0, 0)),
        ],
        out_specs=[
            pl.BlockSpec((1, _N, _CE), lambda b: (b, 0, 0)),
            pl.BlockSpec((1, 1, _N), lambda b: (b, 0, 0)),
            pl.BlockSpec((1, 1), lambda b: (0, 0)),
        ],
        out_shape=[
            jax.ShapeDtypeStruct((batch, _N, _CE), f32),
            jax.ShapeDtypeStruct((batch, 1, _N), f32),
            jax.ShapeDtypeStruct((1, 1), f32),
        ],
        scratch_shapes=[pltpu.VMEM((1, _NC), f32)],
        interpret=interpret,
    )(x0, x1, w0, w1, mup, ct, b2, r2)


def kernel(p0, p1, W_proj, b_proj, centroids, r):
    batch = p0.shape[0]
    x0 = p0.reshape(batch, _C0, _N)
    x1 = p1.reshape(batch, _C1, _N1)
    embeds, score3, loss = _run(
        x0, x1, W_proj[:, :_C0], W_proj[:, _C0:], jnp.asarray(_MUP),
        centroids,
        b_proj.reshape(1, _CE), r.reshape(1, 1))
    score = score3.reshape(batch, 28, 28)[:, None, :, :]
    return (loss[0, 0], score, embeds)
